# Initial kernel scaffold; baseline (speedup 1.0000x reference)
import functools

import jax
import jax.numpy as jnp
from jax import lax
from jax.experimental import pallas as pl
from jax.experimental.pallas import tpu as pltpu
from jax.experimental.pallas import tpu_sc as plsc

_N = 10000
_E = 320000
_D = 128
_H = 2 * _D
_BN_EPS = 1e-5

_NC = 2
_NS = 16
_CHUNK = 128
_TILES = _NC * _NS
_CPT = -(-_E // (_TILES * _CHUNK))
_E_PAD = _TILES * _CPT * _CHUNK
_RPT = _N // _NS

_sc_mesh = plsc.VectorSubcoreMesh(core_axis_name="c", subcore_axis_name="s")


@functools.partial(
    pl.kernel,
    mesh=_sc_mesh,
    out_type=jax.ShapeDtypeStruct((_NC, _N, _D), jnp.float32),
    scratch_types=[
        pltpu.VMEM((_CPT, _CHUNK), jnp.int32),
        pltpu.VMEM((_CPT, _CHUNK), jnp.int32),
        pltpu.VMEM((_CHUNK, _D), jnp.float32),
        pltpu.VMEM_SHARED((_N, _D), jnp.float32),
        pltpu.SemaphoreType.DMA,
    ],
)
def _sc_aggregate(x_hbm, src_hbm, dst_hbm, zero_hbm, out_hbm,
                  src_v, dst_v, rows_v, agg_sh, sem):
    c = lax.axis_index("c")
    s = lax.axis_index("s")
    tid = c * _NS + s

    pltpu.sync_copy(src_hbm.at[tid], src_v)
    pltpu.sync_copy(dst_hbm.at[tid], dst_v)

    pltpu.sync_copy(zero_hbm, agg_sh.at[pl.ds(s * _RPT, _RPT)])
    plsc.subcore_barrier()

    def body(j, carry):
        pltpu.async_copy(x_hbm.at[src_v.at[j]], rows_v, sem).wait()
        pltpu.sync_copy(rows_v, agg_sh.at[dst_v.at[j]], add=True)
        return carry

    lax.fori_loop(0, _CPT, body, 0)

    plsc.subcore_barrier()
    pltpu.sync_copy(agg_sh.at[pl.ds(s * _RPT, _RPT)],
                    out_hbm.at[c, pl.ds(s * _RPT, _RPT)])


_BLK = 1000


def _mlp1_body(eps_ref, x_ref, agg_ref, w1_ref, b1_ref, h1_ref, stats_ref):
    i = pl.program_id(0)
    h0 = (1.0 + eps_ref[0]) * x_ref[...] + agg_ref[0] + agg_ref[1]
    h1 = jnp.dot(h0, w1_ref[...], preferred_element_type=jnp.float32) + b1_ref[...]
    h1_ref[...] = h1

    @pl.when(i == 0)
    def _():
        stats_ref[...] = jnp.zeros_like(stats_ref)

    s0 = jnp.sum(h1, axis=0, keepdims=True)
    s1 = jnp.sum(h1 * h1, axis=0, keepdims=True)
    stats_ref[...] += jnp.concatenate(
        [s0, s1, jnp.zeros((6, _H), jnp.float32)], axis=0)


def _mlp2_body(h1_ref, stats_ref, gamma_ref, beta_ref, w2_ref, b2_ref, out_ref):
    mean = stats_ref[0:1, :] * (1.0 / _N)
    ex2 = stats_ref[1:2, :] * (1.0 / _N)
    var = ex2 - mean * mean
    rstd = lax.rsqrt(var + _BN_EPS)
    scale = gamma_ref[...] * rstd
    shift = beta_ref[...] - mean * scale
    h = jnp.maximum(h1_ref[...] * scale + shift, 0.0)
    out_ref[...] = jnp.dot(h, w2_ref[...],
                           preferred_element_type=jnp.float32) + b2_ref[...]


def kernel(x, edge_index, eps, W1, b1, gamma, beta, W2, b2):
    src = edge_index[0].astype(jnp.int32)
    dst = edge_index[1].astype(jnp.int32)
    pad = _E_PAD - _E
    src_p = jnp.concatenate([src, jnp.full((pad,), _N, jnp.int32)])
    dst_p = jnp.concatenate([dst, jnp.zeros((pad,), jnp.int32)])
    src_p = src_p.reshape(_TILES, _CPT, _CHUNK)
    dst_p = dst_p.reshape(_TILES, _CPT, _CHUNK)
    x_pad = jnp.concatenate([x, jnp.zeros((1, _D), jnp.float32)], axis=0)
    zero = jnp.zeros((_RPT, _D), jnp.float32)

    agg2 = _sc_aggregate(x_pad, src_p, dst_p, zero)

    eps1 = jnp.reshape(eps, (1,)).astype(jnp.float32)
    h1, stats = pl.pallas_call(
        _mlp1_body,
        grid=(_N // _BLK,),
        in_specs=[
            pl.BlockSpec(memory_space=pltpu.SMEM),
            pl.BlockSpec((_BLK, _D), lambda i: (i, 0)),
            pl.BlockSpec((_NC, _BLK, _D), lambda i: (0, i, 0)),
            pl.BlockSpec((_D, _H), lambda i: (0, 0)),
            pl.BlockSpec((1, _H), lambda i: (0, 0)),
        ],
        out_specs=[
            pl.BlockSpec((_BLK, _H), lambda i: (i, 0)),
            pl.BlockSpec((8, _H), lambda i: (0, 0)),
        ],
        out_shape=[
            jax.ShapeDtypeStruct((_N, _H), jnp.float32),
            jax.ShapeDtypeStruct((8, _H), jnp.float32),
        ],
    )(eps1, x, agg2, W1, b1.reshape(1, _H))

    out = pl.pallas_call(
        _mlp2_body,
        grid=(_N // _BLK,),
        in_specs=[
            pl.BlockSpec((_BLK, _H), lambda i: (i, 0)),
            pl.BlockSpec((8, _H), lambda i: (0, 0)),
            pl.BlockSpec((1, _H), lambda i: (0, 0)),
            pl.BlockSpec((1, _H), lambda i: (0, 0)),
            pl.BlockSpec((_H, _D), lambda i: (0, 0)),
            pl.BlockSpec((1, _D), lambda i: (0, 0)),
        ],
        out_specs=pl.BlockSpec((_BLK, _D), lambda i: (i, 0)),
        out_shape=jax.ShapeDtypeStruct((_N, _D), jnp.float32),
    )(h1, stats, gamma.reshape(1, _H), beta.reshape(1, _H),
      W2, b2.reshape(1, _D))

    return out


# speedup vs baseline: 4.7516x; 4.7516x over previous
import functools

import jax
import jax.numpy as jnp
from jax import lax
from jax.experimental import pallas as pl
from jax.experimental.pallas import tpu as pltpu
from jax.experimental.pallas import tpu_sc as plsc

_N = 10000
_E = 320000
_D = 128
_H = 2 * _D
_BN_EPS = 1e-5

_NC = 2
_NS = 16
_CHUNK = 128
_TILES = _NC * _NS
_CPT = -(-_E // (_TILES * _CHUNK))
_E_PAD = _TILES * _CPT * _CHUNK
_N_PAD = 10240
_RPT = _N_PAD // _NS

_sc_mesh = plsc.VectorSubcoreMesh(core_axis_name="c", subcore_axis_name="s")


@functools.partial(
    pl.kernel,
    mesh=_sc_mesh,
    out_type=jax.ShapeDtypeStruct((_NC, _N_PAD, _D), jnp.float32),
    scratch_types=[
        pltpu.VMEM((_CPT, _CHUNK), jnp.int32),
        pltpu.VMEM((_CPT, _CHUNK), jnp.int32),
        pltpu.VMEM((_CHUNK, _D), jnp.float32),
        pltpu.VMEM_SHARED((_N_PAD, _D), jnp.float32),
        pltpu.SemaphoreType.DMA,
    ],
)
def _sc_aggregate(x_hbm, src_hbm, dst_hbm, zero_hbm, out_hbm,
                  src_v, dst_v, rows_v, agg_sh, sem):
    c = lax.axis_index("c")
    s = lax.axis_index("s")
    tid = c * _NS + s

    pltpu.sync_copy(src_hbm.at[tid], src_v)
    pltpu.sync_copy(dst_hbm.at[tid], dst_v)

    pltpu.sync_copy(zero_hbm, agg_sh.at[pl.ds(s * _RPT, _RPT)])
    plsc.subcore_barrier()

    def body(j, carry):
        pltpu.async_copy(x_hbm.at[src_v.at[j]], rows_v, sem).wait()
        pltpu.sync_copy(rows_v, agg_sh.at[dst_v.at[j]], add=True)
        return carry

    lax.fori_loop(0, _CPT, body, 0)

    plsc.subcore_barrier()
    pltpu.sync_copy(agg_sh.at[pl.ds(s * _RPT, _RPT)],
                    out_hbm.at[c, pl.ds(s * _RPT, _RPT)])


_BLK = 1000


def _mlp1_body(eps_ref, x_ref, agg_ref, w1_ref, b1_ref, h1_ref, stats_ref):
    i = pl.program_id(0)
    h0 = (1.0 + eps_ref[0]) * x_ref[...] + agg_ref[0] + agg_ref[1]
    h1 = jnp.dot(h0, w1_ref[...], preferred_element_type=jnp.float32) + b1_ref[...]
    h1_ref[...] = h1

    @pl.when(i == 0)
    def _():
        stats_ref[...] = jnp.zeros_like(stats_ref)

    s0 = jnp.sum(h1, axis=0, keepdims=True)
    s1 = jnp.sum(h1 * h1, axis=0, keepdims=True)
    stats_ref[...] += jnp.concatenate(
        [s0, s1, jnp.zeros((6, _H), jnp.float32)], axis=0)


def _mlp2_body(h1_ref, stats_ref, gamma_ref, beta_ref, w2_ref, b2_ref, out_ref):
    mean = stats_ref[0:1, :] * (1.0 / _N)
    ex2 = stats_ref[1:2, :] * (1.0 / _N)
    var = ex2 - mean * mean
    rstd = lax.rsqrt(var + _BN_EPS)
    scale = gamma_ref[...] * rstd
    shift = beta_ref[...] - mean * scale
    h = jnp.maximum(h1_ref[...] * scale + shift, 0.0)
    out_ref[...] = jnp.dot(h, w2_ref[...],
                           preferred_element_type=jnp.float32) + b2_ref[...]


def kernel(x, edge_index, eps, W1, b1, gamma, beta, W2, b2):
    src = edge_index[0].astype(jnp.int32)
    dst = edge_index[1].astype(jnp.int32)
    pad = _E_PAD - _E
    src_p = jnp.concatenate([src, jnp.full((pad,), _N, jnp.int32)])
    dst_p = jnp.concatenate([dst, jnp.zeros((pad,), jnp.int32)])
    src_p = src_p.reshape(_TILES, _CPT, _CHUNK)
    dst_p = dst_p.reshape(_TILES, _CPT, _CHUNK)
    x_pad = jnp.concatenate([x, jnp.zeros((8, _D), jnp.float32)], axis=0)
    zero = jnp.zeros((_RPT, _D), jnp.float32)

    agg2 = _sc_aggregate(x_pad, src_p, dst_p, zero)

    eps1 = jnp.reshape(eps, (1,)).astype(jnp.float32)
    h1, stats = pl.pallas_call(
        _mlp1_body,
        grid=(_N // _BLK,),
        in_specs=[
            pl.BlockSpec(memory_space=pltpu.SMEM),
            pl.BlockSpec((_BLK, _D), lambda i: (i, 0)),
            pl.BlockSpec((_NC, _BLK, _D), lambda i: (0, i, 0)),
            pl.BlockSpec((_D, _H), lambda i: (0, 0)),
            pl.BlockSpec((1, _H), lambda i: (0, 0)),
        ],
        out_specs=[
            pl.BlockSpec((_BLK, _H), lambda i: (i, 0)),
            pl.BlockSpec((8, _H), lambda i: (0, 0)),
        ],
        out_shape=[
            jax.ShapeDtypeStruct((_N, _H), jnp.float32),
            jax.ShapeDtypeStruct((8, _H), jnp.float32),
        ],
    )(eps1, x, agg2, W1, b1.reshape(1, _H))

    out = pl.pallas_call(
        _mlp2_body,
        grid=(_N // _BLK,),
        in_specs=[
            pl.BlockSpec((_BLK, _H), lambda i: (i, 0)),
            pl.BlockSpec((8, _H), lambda i: (0, 0)),
            pl.BlockSpec((1, _H), lambda i: (0, 0)),
            pl.BlockSpec((1, _H), lambda i: (0, 0)),
            pl.BlockSpec((_H, _D), lambda i: (0, 0)),
            pl.BlockSpec((1, _D), lambda i: (0, 0)),
        ],
        out_specs=pl.BlockSpec((_BLK, _D), lambda i: (i, 0)),
        out_shape=jax.ShapeDtypeStruct((_N, _D), jnp.float32),
    )(h1, stats, gamma.reshape(1, _H), beta.reshape(1, _H),
      W2, b2.reshape(1, _D))

    return out
